# baseline (speedup 1.0000x reference)
import functools
import math

import jax
import jax.numpy as jnp
from jax import lax
from jax.experimental import pallas as pl
from jax.experimental.pallas import tpu as pltpu
from jax.experimental.pallas import tpu_sc as plsc

_EPS = 1e-12


def _logits_body(x_ref, p_ref, lg_ref, pn_ref, *, temp):
    @pl.when(pl.program_id(0) == 0)
    def _normalize_prototypes():
        p = p_ref[...]
        p_norm = jnp.sqrt(jnp.sum(p * p, axis=1, keepdims=True))
        pn_ref[...] = p / jnp.maximum(p_norm, _EPS)

    xb = x_ref[...]
    p_n = pn_ref[...]

    sq = jnp.sum(xb * xb, axis=1, keepdims=True)
    x_norm = jnp.maximum(jnp.sqrt(sq), _EPS)
    xb_n = xb / x_norm

    s = jax.lax.dot_general(
        xb_n, p_n, (((1,), (1,)), ((), ())),
        preferred_element_type=jnp.float32)
    lg_ref[...] = s.T / temp


def _make_sc_select(E, N):
    info = plsc.get_sparse_core_info()
    NC, NS, L = info.num_cores, info.num_subcores, info.num_lanes
    NW = NC * NS
    C = N // NW

    mesh = plsc.VectorSubcoreMesh(core_axis_name="c", subcore_axis_name="s")

    @functools.partial(
        pl.kernel, mesh=mesh,
        out_type=[
            jax.ShapeDtypeStruct((2, N), jnp.int32),
            jax.ShapeDtypeStruct((2, N), jnp.float32),
        ],
        scratch_types=[
            pltpu.VMEM((E, C), jnp.float32),
            pltpu.VMEM((2, C), jnp.int32),
            pltpu.VMEM((2, C), jnp.float32),
        ],
    )
    def sc_select(lg_hbm, e_hbm, w_hbm, lg_v, e_v, w_v):
        wid = lax.axis_index("s") * NC + lax.axis_index("c")
        base = wid * C
        pltpu.sync_copy(lg_hbm.at[:, pl.ds(base, C)], lg_v)

        def group(g, carry):
            tok = g * L
            vs = [lg_v[e, pl.ds(tok, L)] for e in range(E)]
            m1 = vs[0]
            i1 = jnp.zeros((L,), jnp.int32)
            for e in range(1, E):
                c = vs[e] > m1
                m1 = jnp.where(c, vs[e], m1)
                i1 = jnp.where(c, jnp.int32(e), i1)
            m2 = jnp.full((L,), -jnp.inf, jnp.float32)
            i2 = jnp.zeros((L,), jnp.int32)
            for e in range(E):
                c = jnp.logical_and(i1 != e, vs[e] > m2)
                m2 = jnp.where(c, vs[e], m2)
                i2 = jnp.where(c, jnp.int32(e), i2)
            z = jnp.exp(m2 - m1)
            p2 = z / (1.0 + z)
            e_v[0, pl.ds(tok, L)] = i1
            e_v[1, pl.ds(tok, L)] = i2
            w_v[0, pl.ds(tok, L)] = 1.0 - p2
            w_v[1, pl.ds(tok, L)] = p2
            return carry

        for g in range(C // L):
            group(g, 0)

        pltpu.sync_copy(e_v, e_hbm.at[:, pl.ds(base, C)])
        pltpu.sync_copy(w_v, w_hbm.at[:, pl.ds(base, C)])

    return sc_select


@jax.jit
def kernel(x, prototypes):
    B, S, D = x.shape
    E = prototypes.shape[0]
    N = B * S
    temp = math.sqrt(D)
    T = 2048
    xf = x.reshape(N, D)

    NCHUNK = 2
    CN = N // NCHUNK
    sc_select = _make_sc_select(E, CN)
    tc_logits = pl.pallas_call(
        functools.partial(_logits_body, temp=temp),
        grid=(CN // T,),
        in_specs=[
            pl.BlockSpec((T, D), lambda i: (i, 0)),
            pl.BlockSpec((E, D), lambda i: (0, 0)),
        ],
        out_specs=pl.BlockSpec((E, T), lambda i: (0, i)),
        out_shape=jax.ShapeDtypeStruct((E, CN), jnp.float32),
        scratch_shapes=[pltpu.VMEM((E, D), jnp.float32)],
    )
    parts = []
    for c in range(NCHUNK):
        logits_t = tc_logits(
            jax.lax.slice_in_dim(xf, c * CN, (c + 1) * CN, axis=0),
            prototypes)
        parts.append(sc_select(logits_t))

    experts_t = jnp.concatenate([p[0] for p in parts], axis=1)
    weights_t = jnp.concatenate([p[1] for p in parts], axis=1)
    experts = experts_t.T.reshape(B, S, 2)
    weights = weights_t.T.reshape(B, S, 2)
    return experts, weights


# speedup vs baseline: 2.1456x; 2.1456x over previous
import functools
import math

import jax
import jax.numpy as jnp
from jax import lax
from jax.experimental import pallas as pl
from jax.experimental.pallas import tpu as pltpu
from jax.experimental.pallas import tpu_sc as plsc

_EPS = 1e-12


def _logits_body(x_ref, p_ref, lg_ref, pn_ref, *, temp):
    @pl.when(pl.program_id(0) == 0)
    def _normalize_prototypes():
        p = p_ref[...]
        p_norm = jnp.sqrt(jnp.sum(p * p, axis=1, keepdims=True))
        pn_ref[...] = p / jnp.maximum(p_norm, _EPS)

    xb = x_ref[...]
    p_n = pn_ref[...]

    sq = jnp.sum(xb * xb, axis=1, keepdims=True)
    x_norm = jnp.maximum(jnp.sqrt(sq), _EPS)
    xb_n = xb / x_norm

    s = jax.lax.dot_general(
        xb_n, p_n, (((1,), (1,)), ((), ())),
        preferred_element_type=jnp.float32)
    lg_ref[...] = s.T / temp


def _make_sc_select(E, N):
    info = plsc.get_sparse_core_info()
    NC, NS, L = info.num_cores, info.num_subcores, info.num_lanes
    NW = NC * NS
    C = N // NW

    mesh = plsc.VectorSubcoreMesh(core_axis_name="c", subcore_axis_name="s")

    @functools.partial(
        pl.kernel, mesh=mesh,
        out_type=[
            jax.ShapeDtypeStruct((2, N), jnp.int32),
            jax.ShapeDtypeStruct((2, N), jnp.float32),
        ],
        scratch_types=[
            pltpu.VMEM((E, C), jnp.float32),
            pltpu.VMEM((2, C), jnp.int32),
            pltpu.VMEM((2, C), jnp.float32),
        ],
    )
    def sc_select(lg_hbm, e_hbm, w_hbm, lg_v, e_v, w_v):
        wid = lax.axis_index("s") * NC + lax.axis_index("c")
        base = wid * C
        pltpu.sync_copy(lg_hbm.at[:, pl.ds(base, C)], lg_v)

        def group(g, carry):
            tok = g * L
            vs = [lg_v[e, pl.ds(tok, L)] for e in range(E)]
            m1 = vs[0]
            i1 = jnp.zeros((L,), jnp.int32)
            for e in range(1, E):
                c = vs[e] > m1
                m1 = jnp.where(c, vs[e], m1)
                i1 = jnp.where(c, jnp.int32(e), i1)
            m2 = jnp.full((L,), -jnp.inf, jnp.float32)
            i2 = jnp.zeros((L,), jnp.int32)
            for e in range(E):
                c = jnp.logical_and(i1 != e, vs[e] > m2)
                m2 = jnp.where(c, vs[e], m2)
                i2 = jnp.where(c, jnp.int32(e), i2)
            z = jnp.exp(m2 - m1)
            p2 = z / (1.0 + z)
            e_v[0, pl.ds(tok, L)] = i1
            e_v[1, pl.ds(tok, L)] = i2
            w_v[0, pl.ds(tok, L)] = 1.0 - p2
            w_v[1, pl.ds(tok, L)] = p2
            return carry

        for g in range(C // L):
            group(g, 0)

        pltpu.sync_copy(e_v, e_hbm.at[:, pl.ds(base, C)])
        pltpu.sync_copy(w_v, w_hbm.at[:, pl.ds(base, C)])

    return sc_select


@jax.jit
def kernel(x, prototypes):
    B, S, D = x.shape
    E = prototypes.shape[0]
    N = B * S
    temp = math.sqrt(D)
    T = 2048
    xf = x.reshape(N, D)

    NCHUNK = 2
    CN = N // NCHUNK
    sc_select = _make_sc_select(E, CN)
    parts = []
    for c in range(NCHUNK):
        base = c * (CN // T)
        logits_t = pl.pallas_call(
            functools.partial(_logits_body, temp=temp),
            grid=(CN // T,),
            in_specs=[
                pl.BlockSpec((T, D), lambda i, b=base: (b + i, 0)),
                pl.BlockSpec((E, D), lambda i: (0, 0)),
            ],
            out_specs=pl.BlockSpec((E, T), lambda i: (0, i)),
            out_shape=jax.ShapeDtypeStruct((E, CN), jnp.float32),
            scratch_shapes=[pltpu.VMEM((E, D), jnp.float32)],
        )(xf, prototypes)
        parts.append(sc_select(logits_t))

    experts_t = jnp.concatenate([p[0] for p in parts], axis=1)
    weights_t = jnp.concatenate([p[1] for p in parts], axis=1)
    experts = experts_t.T.reshape(B, S, 2)
    weights = weights_t.T.reshape(B, S, 2)
    return experts, weights


# speedup vs baseline: 2.3810x; 1.1097x over previous
import functools
import math

import jax
import jax.numpy as jnp
from jax import lax
from jax.experimental import pallas as pl
from jax.experimental.pallas import tpu as pltpu
from jax.experimental.pallas import tpu_sc as plsc

_EPS = 1e-12


def _logits_body(x_ref, p_ref, lg_ref, pn_ref, *, temp):
    @pl.when(pl.program_id(0) == 0)
    def _normalize_prototypes():
        p = p_ref[...]
        p_norm = jnp.sqrt(jnp.sum(p * p, axis=1, keepdims=True))
        pn_ref[...] = p / jnp.maximum(p_norm, _EPS)

    xb = x_ref[...]
    p_n = pn_ref[...]

    sq = jnp.sum(xb * xb, axis=1, keepdims=True)
    x_norm = jnp.maximum(jnp.sqrt(sq), _EPS)
    xb_n = xb / x_norm

    s = jax.lax.dot_general(
        xb_n, p_n, (((1,), (1,)), ((), ())),
        preferred_element_type=jnp.float32)
    lg_ref[...] = s.T / temp


def _make_sc_select(E, N):
    info = plsc.get_sparse_core_info()
    NC, NS, L = info.num_cores, info.num_subcores, info.num_lanes
    NW = NC * NS
    C = N // NW

    mesh = plsc.VectorSubcoreMesh(core_axis_name="c", subcore_axis_name="s")

    @functools.partial(
        pl.kernel, mesh=mesh,
        out_type=[
            jax.ShapeDtypeStruct((2, N), jnp.int32),
            jax.ShapeDtypeStruct((2, N), jnp.float32),
        ],
        scratch_types=[
            pltpu.VMEM((E, C), jnp.float32),
            pltpu.VMEM((2, C), jnp.int32),
            pltpu.VMEM((2, C), jnp.float32),
        ],
    )
    def sc_select(lg_hbm, e_hbm, w_hbm, lg_v, e_v, w_v):
        wid = lax.axis_index("s") * NC + lax.axis_index("c")
        base = wid * C
        pltpu.sync_copy(lg_hbm.at[:, pl.ds(base, C)], lg_v)

        def group(g, carry):
            tok = g * L
            vs = [lg_v[e, pl.ds(tok, L)] for e in range(E)]
            m1 = vs[0]
            i1 = jnp.zeros((L,), jnp.int32)
            for e in range(1, E):
                c = vs[e] > m1
                m1 = jnp.where(c, vs[e], m1)
                i1 = jnp.where(c, jnp.int32(e), i1)
            m2 = jnp.full((L,), -jnp.inf, jnp.float32)
            i2 = jnp.zeros((L,), jnp.int32)
            for e in range(E):
                c = jnp.logical_and(i1 != e, vs[e] > m2)
                m2 = jnp.where(c, vs[e], m2)
                i2 = jnp.where(c, jnp.int32(e), i2)
            z = jnp.exp(m2 - m1)
            p2 = z / (1.0 + z)
            e_v[0, pl.ds(tok, L)] = i1
            e_v[1, pl.ds(tok, L)] = i2
            w_v[0, pl.ds(tok, L)] = 1.0 - p2
            w_v[1, pl.ds(tok, L)] = p2
            return carry

        for g in range(C // L):
            group(g, 0)

        pltpu.sync_copy(e_v, e_hbm.at[:, pl.ds(base, C)])
        pltpu.sync_copy(w_v, w_hbm.at[:, pl.ds(base, C)])

    return sc_select


@jax.jit
def kernel(x, prototypes):
    B, S, D = x.shape
    E = prototypes.shape[0]
    N = B * S
    temp = math.sqrt(D)
    T = 2048
    xf = x.reshape(N, D)

    logits_t = pl.pallas_call(
        functools.partial(_logits_body, temp=temp),
        grid=(N // T,),
        in_specs=[
            pl.BlockSpec((T, D), lambda i: (i, 0)),
            pl.BlockSpec((E, D), lambda i: (0, 0)),
        ],
        out_specs=pl.BlockSpec((E, T), lambda i: (0, i)),
        out_shape=jax.ShapeDtypeStruct((E, N), jnp.float32),
        scratch_shapes=[pltpu.VMEM((E, D), jnp.float32)],
    )(xf, prototypes)

    experts_t, weights_t = _make_sc_select(E, N)(logits_t)

    experts = experts_t.T.reshape(B, S, 2)
    weights = weights_t.T.reshape(B, S, 2)
    return experts, weights


# speedup vs baseline: 2.3925x; 1.0049x over previous
import functools
import math

import jax
import jax.numpy as jnp
from jax import lax
from jax.experimental import pallas as pl
from jax.experimental.pallas import tpu as pltpu
from jax.experimental.pallas import tpu_sc as plsc

_EPS = 1e-12


def _logits_body(x_ref, p_ref, lg_ref, pn_ref, *, temp):
    @pl.when(pl.program_id(0) == 0)
    def _normalize_prototypes():
        p = p_ref[...]
        p_norm = jnp.sqrt(jnp.sum(p * p, axis=1, keepdims=True))
        pn_ref[...] = p / jnp.maximum(p_norm, _EPS)

    xb = x_ref[...]
    p_n = pn_ref[...]

    sq = jnp.sum(xb * xb, axis=1, keepdims=True)
    x_norm = jnp.maximum(jnp.sqrt(sq), _EPS)
    xb_n = xb / x_norm

    s = jax.lax.dot_general(
        xb_n, p_n, (((1,), (1,)), ((), ())),
        preferred_element_type=jnp.float32)
    lg_ref[...] = s.T / temp


def _make_sc_select(E, N):
    info = plsc.get_sparse_core_info()
    NC, NS, L = info.num_cores, info.num_subcores, info.num_lanes
    NW = NC * NS
    C = N // NW

    mesh = plsc.VectorSubcoreMesh(core_axis_name="c", subcore_axis_name="s")

    @functools.partial(
        pl.kernel, mesh=mesh,
        out_type=[
            jax.ShapeDtypeStruct((2, N), jnp.int32),
            jax.ShapeDtypeStruct((2, N), jnp.float32),
        ],
        scratch_types=[
            pltpu.VMEM((E, C), jnp.float32),
            pltpu.VMEM((2, C), jnp.int32),
            pltpu.VMEM((2, C), jnp.float32),
        ],
    )
    def sc_select(lg_hbm, e_hbm, w_hbm, lg_v, e_v, w_v):
        wid = lax.axis_index("s") * NC + lax.axis_index("c")
        base = wid * C
        pltpu.sync_copy(lg_hbm.at[:, pl.ds(base, C)], lg_v)

        def group(g, carry):
            tok = g * L
            vs = [lg_v[e, pl.ds(tok, L)] for e in range(E)]
            m1 = vs[0]
            i1 = jnp.zeros((L,), jnp.int32)
            for e in range(1, E):
                c = vs[e] > m1
                m1 = jnp.where(c, vs[e], m1)
                i1 = jnp.where(c, jnp.int32(e), i1)
            m2 = jnp.full((L,), -jnp.inf, jnp.float32)
            i2 = jnp.zeros((L,), jnp.int32)
            for e in range(E):
                c = jnp.logical_and(i1 != e, vs[e] > m2)
                m2 = jnp.where(c, vs[e], m2)
                i2 = jnp.where(c, jnp.int32(e), i2)
            z = jnp.exp(m2 - m1)
            p2 = z / (1.0 + z)
            e_v[0, pl.ds(tok, L)] = i1
            e_v[1, pl.ds(tok, L)] = i2
            w_v[0, pl.ds(tok, L)] = 1.0 - p2
            w_v[1, pl.ds(tok, L)] = p2
            return carry

        lax.fori_loop(0, C // L, group, 0)

        pltpu.sync_copy(e_v, e_hbm.at[:, pl.ds(base, C)])
        pltpu.sync_copy(w_v, w_hbm.at[:, pl.ds(base, C)])

    return sc_select


@jax.jit
def kernel(x, prototypes):
    B, S, D = x.shape
    E = prototypes.shape[0]
    N = B * S
    temp = math.sqrt(D)
    T = 2048
    xf = x.reshape(N, D)

    logits_t = pl.pallas_call(
        functools.partial(_logits_body, temp=temp),
        grid=(N // T,),
        in_specs=[
            pl.BlockSpec((T, D), lambda i: (i, 0)),
            pl.BlockSpec((E, D), lambda i: (0, 0)),
        ],
        out_specs=pl.BlockSpec((E, T), lambda i: (0, i)),
        out_shape=jax.ShapeDtypeStruct((E, N), jnp.float32),
        scratch_shapes=[pltpu.VMEM((E, D), jnp.float32)],
    )(xf, prototypes)

    experts_t, weights_t = _make_sc_select(E, N)(logits_t)

    experts = experts_t.T.reshape(B, S, 2)
    weights = weights_t.T.reshape(B, S, 2)
    return experts, weights
